# baseline (speedup 1.0000x reference)
import functools

import jax
import jax.numpy as jnp
from jax import lax
from jax.experimental import pallas as pl
from jax.experimental.pallas import tpu as pltpu

_ALPHA = 1.702
_LIMIT = 7.0


def _moe_kernel(step_group, step_mtile, step_start, step_end, step_first,
                x_ref, gw_ref, uw_ref, gb_ref, ub_ref, dw_ref, db_ref, w_ref,
                out_ref, *, bm):
    s = pl.program_id(0)
    x = x_ref[...]
    gate = jnp.dot(x, gw_ref[0, 0], preferred_element_type=jnp.float32)
    gate = gate + gb_ref[0, 0].astype(jnp.float32)
    up = jnp.dot(x, uw_ref[0, 0], preferred_element_type=jnp.float32)
    up = up + ub_ref[0, 0].astype(jnp.float32)
    gate = jnp.minimum(gate, _LIMIT)
    up = jnp.clip(up, -_LIMIT, _LIMIT)
    glu = gate * jax.nn.sigmoid(gate * _ALPHA)
    inter = ((up + 1.0) * glu).astype(x.dtype)
    out = jnp.dot(inter, dw_ref[0], preferred_element_type=jnp.float32)
    out = (out + db_ref[0, 0].astype(jnp.float32)) * w_ref[:, 0:1]
    row0 = step_mtile[s] * bm
    rows = row0 + lax.broadcasted_iota(jnp.int32, (bm, 1), 0)
    mask = (rows >= step_start[s]) & (rows < step_end[s])
    res = jnp.where(mask, out, 0.0).astype(out_ref.dtype)

    @pl.when(step_first[s] == 1)
    def _():
        out_ref[...] = res

    @pl.when(step_first[s] == 0)
    def _():
        out_ref[...] += res


def _grouped_ffn(x_sorted, w_sorted, gu_cat, gate_b, up_b, down_w,
                 down_b, group_sizes):
    R, H = x_sorted.shape
    E, FF, _ = down_w.shape
    bm = min(512, R)
    num_m = R // bm
    S = num_m + E - 1

    offsets = jnp.cumsum(group_sizes)
    starts = offsets - group_sizes
    first_tile = starts // bm
    last_tile = jnp.where(group_sizes > 0, (offsets - 1) // bm, 0)
    tiles_g = jnp.where(group_sizes > 0, last_tile - first_tile + 1, 0)
    cum_tiles = jnp.cumsum(tiles_g)
    s_arr = jnp.arange(S, dtype=jnp.int32)
    g = jnp.searchsorted(cum_tiles, s_arr, side='right').astype(jnp.int32)
    valid = g < E
    gc = jnp.minimum(g, E - 1)
    prev_cum = jnp.concatenate([jnp.zeros((1,), cum_tiles.dtype), cum_tiles])[gc]
    mtile = jnp.where(valid, first_tile[gc] + (s_arr - prev_cum), num_m - 1)
    step_start = jnp.where(valid, starts[gc], 0).astype(jnp.int32)
    step_end = jnp.where(valid, offsets[gc], 0).astype(jnp.int32)
    mtile = mtile.astype(jnp.int32)
    prev_mtile = jnp.concatenate([jnp.full((1,), -1, jnp.int32), mtile[:-1]])
    step_first = (mtile != prev_mtile).astype(jnp.int32)

    w_col = jnp.broadcast_to(w_sorted[:, None], (R, 128)).astype(jnp.float32)

    grid_spec = pltpu.PrefetchScalarGridSpec(
        num_scalar_prefetch=5,
        grid=(S,),
        in_specs=[
            pl.BlockSpec((bm, H), lambda s, sg, sm, sst, sen, sf: (sm[s], 0)),
            pl.BlockSpec((1, 1, H, FF), lambda s, sg, sm, sst, sen, sf: (sg[s], 0, 0, 0)),
            pl.BlockSpec((1, 1, H, FF), lambda s, sg, sm, sst, sen, sf: (sg[s], 1, 0, 0)),
            pl.BlockSpec((1, 1, FF), lambda s, sg, sm, sst, sen, sf: (sg[s], 0, 0)),
            pl.BlockSpec((1, 1, FF), lambda s, sg, sm, sst, sen, sf: (sg[s], 0, 0)),
            pl.BlockSpec((1, FF, H), lambda s, sg, sm, sst, sen, sf: (sg[s], 0, 0)),
            pl.BlockSpec((1, 1, H), lambda s, sg, sm, sst, sen, sf: (sg[s], 0, 0)),
            pl.BlockSpec((bm, 128), lambda s, sg, sm, sst, sen, sf: (sm[s], 0)),
        ],
        out_specs=pl.BlockSpec((bm, H), lambda s, sg, sm, sst, sen, sf: (sm[s], 0)),
    )
    return pl.pallas_call(
        functools.partial(_moe_kernel, bm=bm),
        grid_spec=grid_spec,
        out_shape=jax.ShapeDtypeStruct((R, H), jnp.bfloat16),
        compiler_params=pltpu.CompilerParams(
            dimension_semantics=("arbitrary",),
            vmem_limit_bytes=100 * 1024 * 1024,
        ),
    )(gc, mtile, step_start, step_end, step_first,
      x_sorted, gu_cat, gu_cat, gate_b[:, None, :], up_b[:, None, :],
      down_w, down_b[:, None, :], w_col)


def kernel(hidden_states, topk_weights, topk_ids, gate_up_proj,
           gate_up_proj_bias, down_proj, down_proj_bias):
    T, H = hidden_states.shape
    E, _, FF2 = gate_up_proj.shape
    top_k = topk_ids.shape[1]
    R = T * top_k

    flat = topk_ids.reshape(-1).astype(jnp.int32)

    oh = (flat[:, None] == jnp.arange(E, dtype=jnp.int32)[None, :]).astype(jnp.int32)
    csum = jnp.cumsum(oh, axis=0)
    rank = jnp.sum(csum * oh, axis=1) - 1
    group_sizes = csum[-1]
    offsets_excl = jnp.concatenate(
        [jnp.zeros((1,), jnp.int32), jnp.cumsum(group_sizes)[:-1]])
    dest = jnp.sum(oh * offsets_excl[None, :], axis=1) + rank
    sorted_sel = jnp.zeros((R,), jnp.int32).at[dest].set(
        jnp.arange(R, dtype=jnp.int32))
    sorted_idx = sorted_sel // top_k

    x_sorted = hidden_states.astype(jnp.bfloat16)[sorted_idx]
    w_sorted = topk_weights.reshape(-1)[sorted_sel]

    FF = FF2 // 2
    gu_bf = gate_up_proj.astype(jnp.bfloat16)
    packed = lax.bitcast_convert_type(gu_bf.reshape(E, H, FF, 2), jnp.uint32)
    gate_w = lax.bitcast_convert_type(
        (packed & 0xFFFF).astype(jnp.uint16), jnp.bfloat16)
    up_w = lax.bitcast_convert_type(
        (packed >> 16).astype(jnp.uint16), jnp.bfloat16)
    gu_cat = jnp.stack([gate_w, up_w], axis=1)
    gate_b = gate_up_proj_bias[:, 0::2]
    up_b = gate_up_proj_bias[:, 1::2]
    down_w = down_proj.astype(jnp.bfloat16)

    out_rows = _grouped_ffn(x_sorted, w_sorted, gu_cat, gate_b, up_b,
                            down_w, down_proj_bias, group_sizes)

    unsorted = out_rows[dest].astype(jnp.float32)
    combined = unsorted.reshape(T, top_k, H).sum(axis=1)
    return combined.astype(hidden_states.dtype)


# speedup vs baseline: 1.0557x; 1.0557x over previous
import functools

import jax
import jax.numpy as jnp
from jax import lax
from jax.experimental import pallas as pl
from jax.experimental.pallas import tpu as pltpu

_ALPHA = 1.702
_LIMIT = 7.0


def _moe_kernel(step_group, step_mtile, step_start, step_end, step_first,
                x_ref, guw_ref, gb_ref, ub_ref, dw_ref, db_ref, w_ref,
                out_ref, *, bm):
    s = pl.program_id(0)
    x = x_ref[...]
    wp = guw_ref[0]
    gate_w = lax.bitcast_convert_type(wp << 16, jnp.float32).astype(x.dtype)
    up_w = lax.bitcast_convert_type(
        wp & jnp.int32(-65536), jnp.float32).astype(x.dtype)
    gate = jnp.dot(x, gate_w, preferred_element_type=jnp.float32)
    gate = gate + gb_ref[0, 0].astype(jnp.float32)
    up = jnp.dot(x, up_w, preferred_element_type=jnp.float32)
    up = up + ub_ref[0, 0].astype(jnp.float32)
    gate = jnp.minimum(gate, _LIMIT)
    up = jnp.clip(up, -_LIMIT, _LIMIT)
    glu = gate * jax.nn.sigmoid(gate * _ALPHA)
    inter = ((up + 1.0) * glu).astype(x.dtype)
    out = jnp.dot(inter, dw_ref[0], preferred_element_type=jnp.float32)
    out = (out + db_ref[0, 0].astype(jnp.float32)) * w_ref[:, 0:1]
    row0 = step_mtile[s] * bm
    rows = row0 + lax.broadcasted_iota(jnp.int32, (bm, 1), 0)
    mask = (rows >= step_start[s]) & (rows < step_end[s])
    res = jnp.where(mask, out, 0.0).astype(out_ref.dtype)

    @pl.when(step_first[s] == 1)
    def _():
        out_ref[...] = res

    @pl.when(step_first[s] == 0)
    def _():
        out_ref[...] += res


def _grouped_ffn(x_sorted, w_sorted, gu_packed, gate_b, up_b, down_w,
                 down_b, group_sizes):
    R, H = x_sorted.shape
    E, FF, _ = down_w.shape
    bm = min(512, R)
    num_m = R // bm
    S = num_m + E - 1

    offsets = jnp.cumsum(group_sizes)
    starts = offsets - group_sizes
    first_tile = starts // bm
    last_tile = jnp.where(group_sizes > 0, (offsets - 1) // bm, 0)
    tiles_g = jnp.where(group_sizes > 0, last_tile - first_tile + 1, 0)
    cum_tiles = jnp.cumsum(tiles_g)
    s_arr = jnp.arange(S, dtype=jnp.int32)
    g = jnp.searchsorted(cum_tiles, s_arr, side='right').astype(jnp.int32)
    valid = g < E
    gc = jnp.minimum(g, E - 1)
    prev_cum = jnp.concatenate([jnp.zeros((1,), cum_tiles.dtype), cum_tiles])[gc]
    mtile = jnp.where(valid, first_tile[gc] + (s_arr - prev_cum), num_m - 1)
    step_start = jnp.where(valid, starts[gc], 0).astype(jnp.int32)
    step_end = jnp.where(valid, offsets[gc], 0).astype(jnp.int32)
    mtile = mtile.astype(jnp.int32)
    prev_mtile = jnp.concatenate([jnp.full((1,), -1, jnp.int32), mtile[:-1]])
    step_first = (mtile != prev_mtile).astype(jnp.int32)

    w_col = jnp.broadcast_to(w_sorted[:, None], (R, 128)).astype(jnp.float32)

    grid_spec = pltpu.PrefetchScalarGridSpec(
        num_scalar_prefetch=5,
        grid=(S,),
        in_specs=[
            pl.BlockSpec((bm, H), lambda s, sg, sm, sst, sen, sf: (sm[s], 0)),
            pl.BlockSpec((1, H, FF), lambda s, sg, sm, sst, sen, sf: (sg[s], 0, 0)),
            pl.BlockSpec((1, 1, FF), lambda s, sg, sm, sst, sen, sf: (sg[s], 0, 0)),
            pl.BlockSpec((1, 1, FF), lambda s, sg, sm, sst, sen, sf: (sg[s], 0, 0)),
            pl.BlockSpec((1, FF, H), lambda s, sg, sm, sst, sen, sf: (sg[s], 0, 0)),
            pl.BlockSpec((1, 1, H), lambda s, sg, sm, sst, sen, sf: (sg[s], 0, 0)),
            pl.BlockSpec((bm, 128), lambda s, sg, sm, sst, sen, sf: (sm[s], 0)),
        ],
        out_specs=pl.BlockSpec((bm, H), lambda s, sg, sm, sst, sen, sf: (sm[s], 0)),
    )
    return pl.pallas_call(
        functools.partial(_moe_kernel, bm=bm),
        grid_spec=grid_spec,
        out_shape=jax.ShapeDtypeStruct((R, H), jnp.bfloat16),
        compiler_params=pltpu.CompilerParams(
            dimension_semantics=("arbitrary",),
            vmem_limit_bytes=100 * 1024 * 1024,
        ),
    )(gc, mtile, step_start, step_end, step_first,
      x_sorted, gu_packed, gate_b[:, None, :], up_b[:, None, :],
      down_w, down_b[:, None, :], w_col)


def kernel(hidden_states, topk_weights, topk_ids, gate_up_proj,
           gate_up_proj_bias, down_proj, down_proj_bias):
    T, H = hidden_states.shape
    E, _, FF2 = gate_up_proj.shape
    top_k = topk_ids.shape[1]
    R = T * top_k

    flat = topk_ids.reshape(-1).astype(jnp.int32)

    oh = (flat[:, None] == jnp.arange(E, dtype=jnp.int32)[None, :]).astype(jnp.int32)
    csum = jnp.cumsum(oh, axis=0)
    rank = jnp.sum(csum * oh, axis=1) - 1
    group_sizes = csum[-1]
    offsets_excl = jnp.concatenate(
        [jnp.zeros((1,), jnp.int32), jnp.cumsum(group_sizes)[:-1]])
    dest = jnp.sum(oh * offsets_excl[None, :], axis=1) + rank
    sorted_sel = jnp.zeros((R,), jnp.int32).at[dest].set(
        jnp.arange(R, dtype=jnp.int32))
    sorted_idx = sorted_sel // top_k

    x_sorted = hidden_states.astype(jnp.bfloat16)[sorted_idx]
    w_sorted = topk_weights.reshape(-1)[sorted_sel]

    FF = FF2 // 2
    gu_bf = gate_up_proj.astype(jnp.bfloat16)
    gu_packed = lax.bitcast_convert_type(
        gu_bf.reshape(E, H, FF2 // 2, 2), jnp.int32)
    gate_b = gate_up_proj_bias[:, 0::2]
    up_b = gate_up_proj_bias[:, 1::2]
    down_w = down_proj.astype(jnp.bfloat16)

    out_rows = _grouped_ffn(x_sorted, w_sorted, gu_packed, gate_b, up_b,
                            down_w, down_proj_bias, group_sizes)

    unsorted = out_rows[dest].astype(jnp.float32)
    combined = unsorted.reshape(T, top_k, H).sum(axis=1)
    return combined.astype(hidden_states.dtype)


# speedup vs baseline: 1.0785x; 1.0215x over previous
import functools

import jax
import jax.numpy as jnp
from jax import lax
from jax.experimental import pallas as pl
from jax.experimental.pallas import tpu as pltpu

_ALPHA = 1.702
_LIMIT = 7.0


def _moe_kernel(step_group, step_mtile, step_start, step_end, step_first,
                x_ref, guw_ref, gb_ref, ub_ref, dw_ref, db_ref,
                out_ref, *, bm):
    s = pl.program_id(0)
    x = x_ref[...]
    wp = guw_ref[0]
    gate_w = lax.bitcast_convert_type(wp << 16, jnp.float32).astype(x.dtype)
    up_w = lax.bitcast_convert_type(
        wp & jnp.int32(-65536), jnp.float32).astype(x.dtype)
    gate = jnp.dot(x, gate_w, preferred_element_type=jnp.float32)
    gate = gate + gb_ref[0, 0].astype(jnp.float32)
    up = jnp.dot(x, up_w, preferred_element_type=jnp.float32)
    up = up + ub_ref[0, 0].astype(jnp.float32)
    gate = jnp.minimum(gate, _LIMIT)
    up = jnp.clip(up, -_LIMIT, _LIMIT)
    glu = gate * jax.nn.sigmoid(gate * _ALPHA)
    inter = ((up + 1.0) * glu).astype(x.dtype)
    out = jnp.dot(inter, dw_ref[0], preferred_element_type=jnp.float32)
    out = out + db_ref[0, 0].astype(jnp.float32)
    row0 = step_mtile[s] * bm
    rows = row0 + lax.broadcasted_iota(jnp.int32, (bm, 1), 0)
    mask = (rows >= step_start[s]) & (rows < step_end[s])
    res = jnp.where(mask, out, 0.0).astype(out_ref.dtype)

    @pl.when(step_first[s] == 1)
    def _():
        out_ref[...] = res

    @pl.when(step_first[s] == 0)
    def _():
        out_ref[...] += res


def _grouped_ffn(x_sorted, gu_packed, gate_b, up_b, down_w,
                 down_b, group_sizes):
    R, H = x_sorted.shape
    E, FF, _ = down_w.shape
    bm = min(512, R)
    num_m = R // bm
    S = num_m + E - 1

    offsets = jnp.cumsum(group_sizes)
    starts = offsets - group_sizes
    first_tile = starts // bm
    last_tile = jnp.where(group_sizes > 0, (offsets - 1) // bm, 0)
    tiles_g = jnp.where(group_sizes > 0, last_tile - first_tile + 1, 0)
    cum_tiles = jnp.cumsum(tiles_g)
    s_arr = jnp.arange(S, dtype=jnp.int32)
    g = jnp.searchsorted(cum_tiles, s_arr, side='right').astype(jnp.int32)
    valid = g < E
    gc = jnp.minimum(g, E - 1)
    prev_cum = jnp.concatenate([jnp.zeros((1,), cum_tiles.dtype), cum_tiles])[gc]
    mtile = jnp.where(valid, first_tile[gc] + (s_arr - prev_cum), num_m - 1)
    step_start = jnp.where(valid, starts[gc], 0).astype(jnp.int32)
    step_end = jnp.where(valid, offsets[gc], 0).astype(jnp.int32)
    mtile = mtile.astype(jnp.int32)
    prev_mtile = jnp.concatenate([jnp.full((1,), -1, jnp.int32), mtile[:-1]])
    step_first = (mtile != prev_mtile).astype(jnp.int32)

    grid_spec = pltpu.PrefetchScalarGridSpec(
        num_scalar_prefetch=5,
        grid=(S,),
        in_specs=[
            pl.BlockSpec((bm, H), lambda s, sg, sm, sst, sen, sf: (sm[s], 0)),
            pl.BlockSpec((1, H, FF), lambda s, sg, sm, sst, sen, sf: (sg[s], 0, 0)),
            pl.BlockSpec((1, 1, FF), lambda s, sg, sm, sst, sen, sf: (sg[s], 0, 0)),
            pl.BlockSpec((1, 1, FF), lambda s, sg, sm, sst, sen, sf: (sg[s], 0, 0)),
            pl.BlockSpec((1, FF, H), lambda s, sg, sm, sst, sen, sf: (sg[s], 0, 0)),
            pl.BlockSpec((1, 1, H), lambda s, sg, sm, sst, sen, sf: (sg[s], 0, 0)),
        ],
        out_specs=pl.BlockSpec((bm, H), lambda s, sg, sm, sst, sen, sf: (sm[s], 0)),
    )
    return pl.pallas_call(
        functools.partial(_moe_kernel, bm=bm),
        grid_spec=grid_spec,
        out_shape=jax.ShapeDtypeStruct((R, H), jnp.bfloat16),
        compiler_params=pltpu.CompilerParams(
            dimension_semantics=("arbitrary",),
            vmem_limit_bytes=100 * 1024 * 1024,
        ),
    )(gc, mtile, step_start, step_end, step_first,
      x_sorted, gu_packed, gate_b[:, None, :], up_b[:, None, :],
      down_w, down_b[:, None, :])


def kernel(hidden_states, topk_weights, topk_ids, gate_up_proj,
           gate_up_proj_bias, down_proj, down_proj_bias):
    T, H = hidden_states.shape
    E, _, FF2 = gate_up_proj.shape
    top_k = topk_ids.shape[1]
    R = T * top_k

    flat = topk_ids.reshape(-1).astype(jnp.int32)

    oh = (flat[:, None] == jnp.arange(E, dtype=jnp.int32)[None, :]).astype(jnp.int32)
    csum = jnp.cumsum(oh, axis=0)
    rank = jnp.sum(csum * oh, axis=1) - 1
    group_sizes = csum[-1]
    offsets_excl = jnp.concatenate(
        [jnp.zeros((1,), jnp.int32), jnp.cumsum(group_sizes)[:-1]])
    dest = jnp.sum(oh * offsets_excl[None, :], axis=1) + rank

    hs_bf = hidden_states.astype(jnp.bfloat16)
    x_src = jnp.repeat(hs_bf, top_k, axis=0)
    x_sorted = jnp.zeros((R, H), jnp.bfloat16).at[dest].set(
        x_src, unique_indices=True)

    FF = FF2 // 2
    gu_bf = gate_up_proj.astype(jnp.bfloat16)
    gu_packed = lax.bitcast_convert_type(
        gu_bf.reshape(E, H, FF2 // 2, 2), jnp.int32)
    gate_b = gate_up_proj_bias[:, 0::2]
    up_b = gate_up_proj_bias[:, 1::2]
    down_w = down_proj.astype(jnp.bfloat16)

    out_rows = _grouped_ffn(x_sorted, gu_packed, gate_b, up_b,
                            down_w, down_proj_bias, group_sizes)

    unsorted = out_rows[dest].reshape(T, top_k, H).astype(jnp.float32)
    combined = jnp.einsum('tkh,tk->th', unsorted,
                          topk_weights.astype(jnp.float32))
    return combined.astype(hidden_states.dtype)
